# baseline (speedup 1.0000x reference)
import functools

import jax
import jax.numpy as jnp
import numpy as np
from jax import lax
from jax.experimental import pallas as pl
from jax.experimental.pallas import tpu as pltpu
from jax.experimental.pallas import tpu_sc as plsc

_N_RET, _N_LGN, _N_V1, _N_IT = 128, 128, 256, 128
_C, _H, _W = 3, 224, 224
_B = 256
_N_CLASSES = 1000
_K = _N_RET * 16
_CHW = _C * _H * _W
_INV_T = 1.25

_NC, _NS = 2, 16
_NW = _NC * _NS
_BPW = _B // _NW
_LANES = 16


_TCAP = 48
_TROWS = _B * _C * _H // 8
_TPS = _C * _H // 8


def _sc_gather(x_tiles, uniq, rpos, sub, col):
    mesh = plsc.VectorSubcoreMesh(core_axis_name="c", subcore_axis_name="s")

    @functools.partial(
        pl.kernel,
        out_type=jax.ShapeDtypeStruct((_B, _K), jnp.float32),
        mesh=mesh,
        compiler_params=pltpu.CompilerParams(use_tc_tiling_on_sc=True,
                                             needs_layout_passes=False),
        scratch_types=[
            pltpu.VMEM((_TCAP,), jnp.int32),
            pltpu.VMEM((_TCAP, 8, _W), jnp.float32),
            pltpu.VMEM((_K,), jnp.int32),
            pltpu.VMEM((_K,), jnp.int32),
            pltpu.VMEM((_K,), jnp.int32),
            pltpu.VMEM((_K,), jnp.float32),
            pltpu.SemaphoreType.DMA,
        ],
    )
    def gather_kernel(x_hbm, uniq_hbm, rpos_hbm, sub_hbm, col_hbm, out_hbm,
                      uniq_v, rows_v, rr_v, ss_v, cc_v, out_v, sem):
        wid = lax.axis_index("s") * _NC + lax.axis_index("c")
        pltpu.sync_copy(uniq_hbm, uniq_v)
        pltpu.sync_copy(rpos_hbm, rr_v)
        pltpu.sync_copy(sub_hbm, ss_v)
        pltpu.sync_copy(col_hbm, cc_v)

        def per_sample(j, _):
            b = wid * _BPW + j
            off = b * _TPS
            copies = []
            for c in range(_TCAP // _LANES):
                vec = uniq_v[pl.ds(c * _LANES, _LANES)] + off
                for l in range(_LANES):
                    i = c * _LANES + l
                    copies.append(pltpu.async_copy(
                        x_hbm.at[vec[l]], rows_v.at[i], sem))
            for cp in copies:
                cp.wait()

            def remap(i, _):
                sl = pl.ds(i * _LANES, _LANES)
                out_v[sl] = plsc.load_gather(
                    rows_v, [rr_v[sl], ss_v[sl], cc_v[sl]])
                return 0

            lax.fori_loop(0, _K // _LANES, remap, 0)
            pltpu.sync_copy(out_v, out_hbm.at[b])
            return 0

        lax.fori_loop(0, _BPW, per_sample, 0)

    return gather_kernel(x_tiles, uniq, rpos, sub, col)


def _dot(a, b):
    return lax.dot_general(a, b, (((1,), (0,)), ((), ())),
                           preferred_element_type=jnp.float32)


def _dot_t(a, b):
    return lax.dot_general(a, b, (((1,), (1,)), ((), ())),
                           preferred_element_type=jnp.float32)


def _tc_body(p_ref, wr_ref, br_ref, swl_ref, bl_ref, swv_ref, bv_ref,
             swi_ref, bi_ref, wct_ref, bc_ref, m1_ref, m2_ref,
             g1_ref, g2_ref, g3_ref, out_ref):
    g1 = g1_ref[...]
    g2 = g2_ref[...]
    g3 = g3_ref[...]

    t = p_ref[...] * wr_ref[...]
    dend = jnp.tanh(_dot(t, g1) + br_ref[...])
    r1 = jax.nn.sigmoid(_dot(dend, g2))

    rep = _dot_t(r1, g2)
    dend = jnp.tanh(rep * swl_ref[...] + bl_ref[...])
    r2 = jax.nn.sigmoid(_dot(dend, g2))

    m1 = m1_ref[...]
    eff1 = jnp.where(jnp.sum(m1, axis=1, keepdims=True) > 0.5, m1, 1.0)
    e = jnp.exp(r2 * _INV_T)
    v1 = _dot_t(e * r2, eff1) / _dot_t(e, eff1)

    rep = _dot_t(v1, g3)
    dend = jnp.tanh(rep * swv_ref[...] + bv_ref[...])
    r3 = jax.nn.sigmoid(_dot(dend, g3))

    m2 = m2_ref[...]
    eff2 = jnp.where(jnp.sum(m2, axis=1, keepdims=True) > 0.5, m2, 1.0)
    e = jnp.exp(r3 * _INV_T)
    it = _dot_t(e * r3, eff2) / _dot_t(e, eff2)

    rep = _dot_t(it, g2)
    dend = jnp.tanh(rep * swi_ref[...] + bi_ref[...])
    r4 = jax.nn.sigmoid(_dot(dend, g2))

    out_ref[...] = _dot(r4, wct_ref[...]) + bc_ref[...]


def _group_mat(n_in, n_out):
    g = np.zeros((n_in, n_out), dtype=np.float32)
    g[np.arange(n_in), np.arange(n_in) // (n_in // n_out)] = 1.0
    return jnp.asarray(g)


def kernel(x, w_retina, b_retina, w_lgn, b_lgn, w_v1, b_v1, w_it, b_it,
           Wc, bc, pixel_map, lgn_to_v1, v1_to_it):
    pmf = pixel_map.reshape(-1)
    rowid = pmf // _W
    tile = rowid // 8
    uniq = jnp.unique(tile, size=_TCAP, fill_value=_TPS - 1)
    rpos = jnp.searchsorted(uniq, tile).astype(jnp.int32)
    p = _sc_gather(x.reshape(_TROWS, 8, _W), uniq.astype(jnp.int32), rpos,
                   (rowid % 8).astype(jnp.int32), (pmf % _W).astype(jnp.int32))

    wr = w_retina.reshape(1, _K)
    br = b_retina.reshape(1, 4 * _N_RET)
    swl = w_lgn.sum(-1).reshape(1, 4 * _N_LGN)
    bl = b_lgn.reshape(1, 4 * _N_LGN)
    swv = w_v1.sum(-1).reshape(1, 4 * _N_V1)
    bv = b_v1.reshape(1, 4 * _N_V1)
    swi = w_it.sum(-1).reshape(1, 4 * _N_IT)
    bi = b_it.reshape(1, 4 * _N_IT)
    wct = Wc.T
    bcr = bc.reshape(1, _N_CLASSES)
    m1 = lgn_to_v1.astype(jnp.float32)
    m2 = v1_to_it.astype(jnp.float32)
    g1 = _group_mat(_K, 4 * _N_RET)
    g2 = _group_mat(4 * _N_LGN, _N_LGN)
    g3 = _group_mat(4 * _N_V1, _N_V1)

    return pl.pallas_call(
        _tc_body,
        out_shape=jax.ShapeDtypeStruct((_B, _N_CLASSES), jnp.float32),
    )(p, wr, br, swl, bl, swv, bv, swi, bi, wct, bcr, m1, m2, g1, g2, g3)


# speedup vs baseline: 12.8773x; 12.8773x over previous
import functools

import jax
import jax.numpy as jnp
import numpy as np
from jax import lax
from jax.experimental import pallas as pl
from jax.experimental.pallas import tpu as pltpu
from jax.experimental.pallas import tpu_sc as plsc

_N_RET, _N_LGN, _N_V1, _N_IT = 128, 128, 256, 128
_C, _H, _W = 3, 224, 224
_B = 256
_N_CLASSES = 1000
_K = _N_RET * 16
_CHW = _C * _H * _W
_INV_T = 1.25

_NC, _NS = 2, 16
_NW = _NC * _NS
_RPW = _K // _NW


def _sc_gather(x_cols, pm_flat):
    mesh = plsc.VectorSubcoreMesh(core_axis_name="c", subcore_axis_name="s")

    @functools.partial(
        pl.kernel,
        out_type=jax.ShapeDtypeStruct((_K, _B), jnp.float32),
        mesh=mesh,
        scratch_types=[
            pltpu.VMEM((_RPW,), jnp.int32),
            pltpu.VMEM((_RPW, _B), jnp.float32),
            pltpu.SemaphoreType.DMA,
        ],
    )
    def gather_kernel(x_hbm, pm_hbm, out_hbm, idx_v, rows_v, sem):
        wid = lax.axis_index("s") * _NC + lax.axis_index("c")
        sl = pl.ds(wid * _RPW, _RPW)
        pltpu.sync_copy(pm_hbm.at[sl], idx_v)
        pltpu.async_copy(x_hbm.at[idx_v], rows_v, sem).wait()
        pltpu.sync_copy(rows_v, out_hbm.at[sl])

    return gather_kernel(x_cols, pm_flat)


def _dot(a, b, dims):
    return lax.dot_general(a, b, (dims, ((), ())),
                           preferred_element_type=jnp.float32)


def _tc_body(p_ref, wr_ref, br_ref, swl_ref, bl_ref, swv_ref, bv_ref,
             swi_ref, bi_ref, wc_ref, bc_ref, m1_ref, m2_ref,
             g1_ref, g2_ref, g3_ref, out_ref):
    g1 = g1_ref[...]
    g2 = g2_ref[...]
    g3 = g3_ref[...]

    t = p_ref[...] * wr_ref[...]
    dend = jnp.tanh(_dot(g1, t, ((0,), (0,))) + br_ref[...])
    r1 = jax.nn.sigmoid(_dot(g2, dend, ((0,), (0,))))

    rep = _dot(g2, r1, ((1,), (0,)))
    dend = jnp.tanh(rep * swl_ref[...] + bl_ref[...])
    r2 = jax.nn.sigmoid(_dot(g2, dend, ((0,), (0,))))

    m1 = m1_ref[...]
    eff1 = jnp.where(jnp.sum(m1, axis=1, keepdims=True) > 0.5, m1, 1.0)
    e = jnp.exp(r2 * _INV_T)
    v1 = (_dot(eff1, e * r2, ((1,), (0,)))
          / _dot(eff1, e, ((1,), (0,))))

    rep = _dot(g3, v1, ((1,), (0,)))
    dend = jnp.tanh(rep * swv_ref[...] + bv_ref[...])
    r3 = jax.nn.sigmoid(_dot(g3, dend, ((0,), (0,))))

    m2 = m2_ref[...]
    eff2 = jnp.where(jnp.sum(m2, axis=1, keepdims=True) > 0.5, m2, 1.0)
    e = jnp.exp(r3 * _INV_T)
    it = (_dot(eff2, e * r3, ((1,), (0,)))
          / _dot(eff2, e, ((1,), (0,))))

    rep = _dot(g2, it, ((1,), (0,)))
    dend = jnp.tanh(rep * swi_ref[...] + bi_ref[...])
    r4 = jax.nn.sigmoid(_dot(g2, dend, ((0,), (0,))))

    out_ref[...] = _dot(r4, wc_ref[...], ((0,), (1,))) + bc_ref[...]


def _group_mat(n_in, n_out):
    g = np.zeros((n_in, n_out), dtype=np.float32)
    g[np.arange(n_in), np.arange(n_in) // (n_in // n_out)] = 1.0
    return jnp.asarray(g)


def kernel(x, w_retina, b_retina, w_lgn, b_lgn, w_v1, b_v1, w_it, b_it,
           Wc, bc, pixel_map, lgn_to_v1, v1_to_it):
    x_cols = x.transpose(1, 2, 3, 0).reshape(_CHW, _B)
    p = _sc_gather(x_cols, pixel_map.reshape(-1))

    wr = w_retina.reshape(_K, 1)
    br = b_retina.reshape(4 * _N_RET, 1)
    swl = w_lgn.sum(-1).reshape(4 * _N_LGN, 1)
    bl = b_lgn.reshape(4 * _N_LGN, 1)
    swv = w_v1.sum(-1).reshape(4 * _N_V1, 1)
    bv = b_v1.reshape(4 * _N_V1, 1)
    swi = w_it.sum(-1).reshape(4 * _N_IT, 1)
    bi = b_it.reshape(4 * _N_IT, 1)
    bcr = bc.reshape(1, _N_CLASSES)
    m1 = lgn_to_v1.astype(jnp.float32)
    m2 = v1_to_it.astype(jnp.float32)
    g1 = _group_mat(_K, 4 * _N_RET)
    g2 = _group_mat(4 * _N_LGN, _N_LGN)
    g3 = _group_mat(4 * _N_V1, _N_V1)

    return pl.pallas_call(
        _tc_body,
        out_shape=jax.ShapeDtypeStruct((_B, _N_CLASSES), jnp.float32),
    )(p, wr, br, swl, bl, swv, bv, swi, bi, Wc, bcr, m1, m2, g1, g2, g3)


# speedup vs baseline: 13.6058x; 1.0566x over previous
import functools

import jax
import jax.numpy as jnp
import numpy as np
from jax import lax
from jax.experimental import pallas as pl
from jax.experimental.pallas import tpu as pltpu
from jax.experimental.pallas import tpu_sc as plsc

_N_RET, _N_LGN, _N_V1, _N_IT = 128, 128, 256, 128
_C, _H, _W = 3, 224, 224
_B = 256
_N_CLASSES = 1000
_K = _N_RET * 16
_CHW = _C * _H * _W
_INV_T = 1.25

_NC, _NS = 2, 16
_NW = _NC * _NS
_RPW = _K // _NW


def _sc_gather(x_cols, pm_flat):
    mesh = plsc.VectorSubcoreMesh(core_axis_name="c", subcore_axis_name="s")

    @functools.partial(
        pl.kernel,
        out_type=jax.ShapeDtypeStruct((_K, _B), jnp.float32),
        mesh=mesh,
        scratch_types=[
            pltpu.VMEM((_RPW,), jnp.int32),
            pltpu.VMEM((_RPW, _B), jnp.float32),
            pltpu.SemaphoreType.DMA,
        ],
    )
    def gather_kernel(x_hbm, pm_hbm, out_hbm, idx_v, rows_v, sem):
        wid = lax.axis_index("s") * _NC + lax.axis_index("c")
        sl = pl.ds(wid * _RPW, _RPW)
        pltpu.sync_copy(pm_hbm.at[sl], idx_v)
        pltpu.async_copy(x_hbm.at[idx_v], rows_v, sem).wait()
        pltpu.sync_copy(rows_v, out_hbm.at[sl])

    return gather_kernel(x_cols, pm_flat)


def _dot(a, b, dims):
    return lax.dot_general(a, b, (dims, ((), ())),
                           preferred_element_type=jnp.float32)


def _soma(inp, w, b):
    sw = jnp.sum(w, axis=-1)
    dend = jnp.tanh(inp[:, None, :] * sw[:, :, None] + b[:, :, None])
    return jax.nn.sigmoid(jnp.sum(dend, axis=1))


def _pool(r, m):
    eff = jnp.where(jnp.sum(m, axis=1, keepdims=True) > 0.5, m, 1.0)
    e = jnp.exp(r * _INV_T)
    return (_dot(eff, e * r, ((1,), (0,)))
            / _dot(eff, e, ((1,), (0,))))


def _tc_body(p_ref, wr_ref, br_ref, wl_ref, bl_ref, wv_ref, bv_ref,
             wi_ref, bi_ref, wc_ref, bc_ref, m1_ref, m2_ref, out_ref):
    p4 = p_ref[...].reshape(_N_RET, 4, 4, _B)
    t = p4 * wr_ref[...][:, :, :, None]
    dend = jnp.tanh(jnp.sum(t, axis=2) + br_ref[...][:, :, None])
    r1 = jax.nn.sigmoid(jnp.sum(dend, axis=1))

    r2 = _soma(r1, wl_ref[...], bl_ref[...])
    v1 = _pool(r2, m1_ref[...])
    r3 = _soma(v1, wv_ref[...], bv_ref[...])
    it = _pool(r3, m2_ref[...])
    r4 = _soma(it, wi_ref[...], bi_ref[...])

    out_ref[...] = _dot(r4, wc_ref[...], ((0,), (1,))) + bc_ref[...]


def kernel(x, w_retina, b_retina, w_lgn, b_lgn, w_v1, b_v1, w_it, b_it,
           Wc, bc, pixel_map, lgn_to_v1, v1_to_it):
    x_cols = x.transpose(1, 2, 3, 0).reshape(_CHW, _B)
    p = _sc_gather(x_cols, pixel_map.reshape(-1))

    m1 = lgn_to_v1.astype(jnp.float32)
    m2 = v1_to_it.astype(jnp.float32)

    return pl.pallas_call(
        _tc_body,
        out_shape=jax.ShapeDtypeStruct((_B, _N_CLASSES), jnp.float32),
    )(p, w_retina, b_retina, w_lgn, b_lgn, w_v1, b_v1, w_it, b_it,
      Wc, bc.reshape(1, _N_CLASSES), m1, m2)
